# Initial kernel scaffold; baseline (speedup 1.0000x reference)
import jax
import jax.numpy as jnp
from jax.experimental import pallas as pl


def kernel(h_V_s, h_V_v, edge_index, h_E_s, h_E_v, batch, params):
    raise NotImplementedError("write your pallas kernel here")


# speedup vs baseline: 7.2257x; 7.2257x over previous
import functools

import jax
import jax.numpy as jnp
from jax import lax
from jax.experimental import pallas as pl
from jax.experimental.pallas import tpu as pltpu
from jax.experimental.pallas import tpu_sc as plsc

ROW_T = 160
ROW_M = 176
NW = 32
GCH = 80
BN = 1000
BE = 512


def _ln(s, g, b):
    mu = jnp.mean(s, axis=1, keepdims=True)
    var = jnp.mean((s - mu) * (s - mu), axis=1, keepdims=True)
    return (s - mu) / jnp.sqrt(var + 1e-5) * g + b


def _vln(v):
    sq = jnp.maximum(v[0] * v[0] + v[1] * v[1] + v[2] * v[2], 1e-8)
    vn = jnp.sqrt(jnp.mean(sq, axis=1, keepdims=True))
    return [vd / vn for vd in v]


def _sigmoid(x):
    return 1.0 / (1.0 + jnp.exp(-x))


def _gvp(s, v, whT, wsT, bs, wvT, relu_s, gate_v):
    if v is not None:
        vh = [vd @ whT for vd in v]
        vn = jnp.sqrt(jnp.maximum(vh[0] * vh[0] + vh[1] * vh[1] + vh[2] * vh[2], 1e-8))
        s = jnp.concatenate([s, vn], axis=1) @ wsT + bs
        if wvT is not None:
            v = [vhd @ wvT for vhd in vh]
            if gate_v:
                nr = jnp.sqrt(jnp.maximum(v[0] * v[0] + v[1] * v[1] + v[2] * v[2], 1e-8))
                gt = _sigmoid(nr)
                v = [vd * gt for vd in v]
        else:
            v = None
    else:
        s = s @ wsT + bs
        v = None
    if relu_s:
        s = jnp.maximum(s, 0.0)
    return s, v


def _unpack_row(t):
    s = t[:, 0:100]
    v = [t[:, 100 + 16 * d:116 + 16 * d] for d in range(3)]
    return s, v


def _pack_row(s, v, width):
    b = s.shape[0]
    parts = [s, v[0], v[1], v[2], jnp.zeros((b, 12), jnp.float32)]
    if width == ROW_M:
        parts.append(jnp.ones((b, 16), jnp.float32))
    return jnp.concatenate(parts, axis=1)


def _tc_map(fn, n_rows, blk, blocked, consts, out_cols):
    grid = (n_rows // blk,)
    in_specs = ([pl.BlockSpec((blk, a.shape[1]), lambda i: (i, 0)) for a in blocked]
                + [pl.BlockSpec(c.shape, lambda i: (0, 0)) for c in consts])
    out_specs = [pl.BlockSpec((blk, oc), lambda i: (i, 0)) for oc in out_cols]
    out_shape = [jax.ShapeDtypeStruct((n_rows, oc), jnp.float32) for oc in out_cols]
    res = pl.pallas_call(fn, grid=grid, in_specs=in_specs,
                         out_specs=out_specs, out_shape=out_shape)(*blocked, *consts)
    return res


def _embed_nodes(h_s, h_v9, params):
    lnp, w = params["Wv_ln"], params["Wv"]
    consts = (lnp["g"][None, :], lnp["b"][None, :], w["wh"].T, w["ws"].T,
              w["bs"][None, :], w["wv"].T)

    def fn(hs, hv, g, b, whT, wsT, bs, wvT, out):
        s = _ln(hs[...], g[...], b[...])
        v = _vln([hv[:, 3 * d:3 * d + 3] for d in range(3)])
        s, v = _gvp(s, v, whT[...], wsT[...], bs[...], wvT[...], False, False)
        out[...] = _pack_row(s, v, ROW_T)

    return _tc_map(fn, h_s.shape[0], BN, [h_s, h_v9], consts, [ROW_T])[0]


def _embed_edges(h_s, ev3, params):
    lnp, w = params["We_ln"], params["We"]
    consts = (lnp["g"][None, :], lnp["b"][None, :], w["wh"].T, w["ws"].T,
              w["bs"][None, :], w["wv"].T)

    def fn(hs, hv, g, b, whT, wsT, bs, wvT, es_out, ev_out):
        s = _ln(hs[...], g[...], b[...])
        v = _vln([hv[:, d:d + 1] for d in range(3)])
        s, v = _gvp(s, v, whT[...], wsT[...], bs[...], wvT[...], False, False)
        es_out[...] = s
        ev_out[...] = jnp.concatenate(
            [v[0], v[1], v[2], jnp.zeros((s.shape[0], 5), jnp.float32)], axis=1)

    es, ev8 = _tc_map(fn, h_s.shape[0], BE, [h_s, ev3], consts, [32, 8])
    return es, ev8


def _tc_messages(gs, gd, es, ev8, lp):
    consts = []
    for m in ("m0", "m1", "m2"):
        consts += [lp[m]["wh"].T, lp[m]["ws"].T, lp[m]["bs"][None, :], lp[m]["wv"].T]
    consts = tuple(consts)

    def fn(gs_r, gd_r, es_r, ev_r, whT0, wsT0, bs0, wvT0, whT1, wsT1, bs1, wvT1,
           whT2, wsT2, bs2, wvT2, out):
        ss, vs = _unpack_row(gs_r[...])
        sd, vd = _unpack_row(gd_r[...])
        es_b = es_r[...]
        ev = ev_r[...]
        ms = jnp.concatenate([ss, es_b, sd], axis=1)
        mv = [jnp.concatenate([vs[d], ev[:, d:d + 1], vd[d]], axis=1) for d in range(3)]
        ms, mv = _gvp(ms, mv, whT0[...], wsT0[...], bs0[...], wvT0[...], True, True)
        ms, mv = _gvp(ms, mv, whT1[...], wsT1[...], bs1[...], wvT1[...], True, True)
        ms, mv = _gvp(ms, mv, whT2[...], wsT2[...], bs2[...], wvT2[...], False, False)
        out[...] = _pack_row(ms, mv, ROW_M)

    return _tc_map(fn, gs.shape[0], BE, [gs, gd, es, ev8], consts, [ROW_M])[0]


def _tc_node_update(t, d0, d1, lp):
    consts = (lp["n0"]["g"][None, :], lp["n0"]["b"][None, :],
              lp["f0"]["wh"].T, lp["f0"]["ws"].T, lp["f0"]["bs"][None, :], lp["f0"]["wv"].T,
              lp["f1"]["wh"].T, lp["f1"]["ws"].T, lp["f1"]["bs"][None, :], lp["f1"]["wv"].T,
              lp["n1"]["g"][None, :], lp["n1"]["b"][None, :])

    def fn(t_r, d0_r, d1_r, g0, b0, whT0, wsT0, bs0, wvT0, whT1, wsT1, bs1, wvT1,
           g1, b1, out):
        s, v = _unpack_row(t_r[...])
        dd0 = d0_r[...]
        dd1 = d1_r[...]
        cnt = jnp.maximum(dd0[:, 160:161] + dd1[:, 160:161], 1.0)
        inv = 1.0 / cnt
        s = s + (dd0[:, 0:100] + dd1[:, 0:100]) * inv
        v = [v[d] + (dd0[:, 100 + 16 * d:116 + 16 * d]
                     + dd1[:, 100 + 16 * d:116 + 16 * d]) * inv for d in range(3)]
        s = _ln(s, g0[...], b0[...])
        v = _vln(v)
        fs, fv = _gvp(s, v, whT0[...], wsT0[...], bs0[...], wvT0[...], True, True)
        fs, fv = _gvp(fs, fv, whT1[...], wsT1[...], bs1[...], wvT1[...], False, False)
        s = _ln(s + fs, g1[...], b1[...])
        v = _vln([v[d] + fv[d] for d in range(3)])
        out[...] = _pack_row(s, v, ROW_T)

    return _tc_map(fn, t.shape[0], BN, [t, d0, d1], consts, [ROW_T])[0]


def _tc_out(t, params):
    lnp, w = params["Wout_ln"], params["Wout"]
    consts = (lnp["g"][None, :], lnp["b"][None, :], w["wh"].T, w["ws"].T,
              w["bs"][None, :])

    def fn(t_r, g, b, whT, wsT, bs, out):
        s, v = _unpack_row(t_r[...])
        s = _ln(s, g[...], b[...])
        v = _vln(v)
        s, _ = _gvp(s, v, whT[...], wsT[...], bs[...], None, True, False)
        out[...] = s

    return _tc_map(fn, t.shape[0], BN, [t], consts, [100])[0]


def _sc_gather_body(table_h, src_h, dst_h, gs_h, gd_h, si_v, di_v, bs_v, bd_v,
                    s_sem, d_sem, *, per_w, n_ch):
    wid = lax.axis_index("s") * 2 + lax.axis_index("c")
    base0 = wid * per_w

    def body(i, carry):
        base = base0 + i * GCH
        pltpu.sync_copy(src_h.at[pl.ds(base, GCH)], si_v)
        pltpu.sync_copy(dst_h.at[pl.ds(base, GCH)], di_v)
        cs = pltpu.async_copy(table_h.at[si_v], bs_v, s_sem)
        cd = pltpu.async_copy(table_h.at[di_v], bd_v, d_sem)
        cs.wait()
        cd.wait()
        pltpu.sync_copy(bs_v, gs_h.at[pl.ds(base, GCH)])
        pltpu.sync_copy(bd_v, gd_h.at[pl.ds(base, GCH)])
        return carry

    lax.fori_loop(0, n_ch, body, 0)


def _sc_gather(table, src, dst):
    e = src.shape[0]
    per_w = e // NW
    n_ch = per_w // GCH
    mesh = plsc.VectorSubcoreMesh(core_axis_name="c", subcore_axis_name="s")
    kfn = pl.kernel(
        functools.partial(_sc_gather_body, per_w=per_w, n_ch=n_ch),
        mesh=mesh,
        out_type=[jax.ShapeDtypeStruct((e, ROW_T), jnp.float32),
                  jax.ShapeDtypeStruct((e, ROW_T), jnp.float32)],
        scratch_types=[pltpu.VMEM((GCH,), jnp.int32),
                       pltpu.VMEM((GCH,), jnp.int32),
                       pltpu.VMEM((GCH, ROW_T), jnp.float32),
                       pltpu.VMEM((GCH, ROW_T), jnp.float32),
                       pltpu.SemaphoreType.DMA,
                       pltpu.SemaphoreType.DMA],
        compiler_params=pltpu.CompilerParams(use_tc_tiling_on_sc=False),
    )
    return kfn(table, src, dst)


def _sc_scatter_body(msgs_h, dst_h, z_h, out_h, buf_v, di_v, acc_s,
                     *, per_w, n_ch, rps):
    cid = lax.axis_index("c")
    sid = lax.axis_index("s")
    wid = sid * 2 + cid
    pltpu.sync_copy(z_h.at[pl.ds(sid * rps, rps)], acc_s.at[pl.ds(sid * rps, rps)])
    plsc.subcore_barrier()
    base0 = wid * per_w

    def body(i, carry):
        base = base0 + i * GCH
        pltpu.sync_copy(msgs_h.at[pl.ds(base, GCH)], buf_v)
        pltpu.sync_copy(dst_h.at[pl.ds(base, GCH)], di_v)
        pltpu.sync_copy(buf_v, acc_s.at[di_v], add=True)
        return carry

    lax.fori_loop(0, n_ch, body, 0)
    plsc.subcore_barrier()
    pltpu.sync_copy(acc_s.at[pl.ds(sid * rps, rps)],
                    out_h.at[cid, pl.ds(sid * rps, rps)])


def _sc_scatter(msgs, dst, zrows):
    e = msgs.shape[0]
    n = zrows.shape[0]
    per_w = e // NW
    n_ch = per_w // GCH
    rps = n // 16
    mesh = plsc.VectorSubcoreMesh(core_axis_name="c", subcore_axis_name="s")
    kfn = pl.kernel(
        functools.partial(_sc_scatter_body, per_w=per_w, n_ch=n_ch, rps=rps),
        mesh=mesh,
        out_type=jax.ShapeDtypeStruct((2, n, ROW_M), jnp.float32),
        scratch_types=[pltpu.VMEM((GCH, ROW_M), jnp.float32),
                       pltpu.VMEM((GCH,), jnp.int32),
                       pltpu.VMEM_SHARED((n, ROW_M), jnp.float32)],
        compiler_params=pltpu.CompilerParams(use_tc_tiling_on_sc=False),
    )
    return kfn(msgs, dst, zrows)


def kernel(h_V_s, h_V_v, edge_index, h_E_s, h_E_v, batch, params):
    n = h_V_s.shape[0]
    src = edge_index[0]
    dst = edge_index[1]
    hv9 = jnp.swapaxes(h_V_v, 1, 2).reshape(n, 9)
    ev3 = h_E_v[:, 0, :]

    t = _embed_nodes(h_V_s, hv9, params)
    es, ev8 = _embed_edges(h_E_s, ev3, params)
    zrows = jnp.zeros((n, ROW_M), jnp.float32)

    for lp in params["layers"]:
        gs, gd = _sc_gather(t, src, dst)
        m = _tc_messages(gs, gd, es, ev8, lp)
        d = _sc_scatter(m, dst, zrows)
        t = _tc_node_update(t, d[0], d[1], lp)

    return _tc_out(t, params)


# speedup vs baseline: 8.1961x; 1.1343x over previous
import functools

import jax
import jax.numpy as jnp
from jax import lax
from jax.experimental import pallas as pl
from jax.experimental.pallas import tpu as pltpu
from jax.experimental.pallas import tpu_sc as plsc

ROW_T = 160
ROW_M = 176
NW = 32
GCH = 128
SCH = 80
BN = 1000
BE = 1280


def _ln(s, g, b):
    mu = jnp.mean(s, axis=1, keepdims=True)
    var = jnp.mean((s - mu) * (s - mu), axis=1, keepdims=True)
    return (s - mu) / jnp.sqrt(var + 1e-5) * g + b


def _vln(v):
    sq = jnp.maximum(v[0] * v[0] + v[1] * v[1] + v[2] * v[2], 1e-8)
    vn = jnp.sqrt(jnp.mean(sq, axis=1, keepdims=True))
    return [vd / vn for vd in v]


def _sigmoid(x):
    return 1.0 / (1.0 + jnp.exp(-x))


def _gvp(s, v, whT, wsT, bs, wvT, relu_s, gate_v):
    if v is not None:
        vh = [vd @ whT for vd in v]
        vn = jnp.sqrt(jnp.maximum(vh[0] * vh[0] + vh[1] * vh[1] + vh[2] * vh[2], 1e-8))
        s = jnp.concatenate([s, vn], axis=1) @ wsT + bs
        if wvT is not None:
            v = [vhd @ wvT for vhd in vh]
            if gate_v:
                nr = jnp.sqrt(jnp.maximum(v[0] * v[0] + v[1] * v[1] + v[2] * v[2], 1e-8))
                gt = _sigmoid(nr)
                v = [vd * gt for vd in v]
        else:
            v = None
    else:
        s = s @ wsT + bs
        v = None
    if relu_s:
        s = jnp.maximum(s, 0.0)
    return s, v


def _unpack_row(t):
    s = t[:, 0:100]
    v = [t[:, 100 + 16 * d:116 + 16 * d] for d in range(3)]
    return s, v


def _pack_row(s, v, width):
    b = s.shape[0]
    parts = [s, v[0], v[1], v[2], jnp.zeros((b, 12), jnp.float32)]
    if width == ROW_M:
        parts.append(jnp.ones((b, 16), jnp.float32))
    return jnp.concatenate(parts, axis=1)


def _tc_map(fn, n_rows, blk, blocked, consts, out_cols):
    grid = (n_rows // blk,)
    in_specs = ([pl.BlockSpec((blk, a.shape[1]), lambda i: (i, 0)) for a in blocked]
                + [pl.BlockSpec(c.shape, lambda i: (0, 0)) for c in consts])
    out_specs = [pl.BlockSpec((blk, oc), lambda i: (i, 0)) for oc in out_cols]
    out_shape = [jax.ShapeDtypeStruct((n_rows, oc), jnp.float32) for oc in out_cols]
    res = pl.pallas_call(fn, grid=grid, in_specs=in_specs,
                         out_specs=out_specs, out_shape=out_shape)(*blocked, *consts)
    return res


def _embed_nodes(h_s, h_v9, params):
    lnp, w = params["Wv_ln"], params["Wv"]
    consts = (lnp["g"][None, :], lnp["b"][None, :], w["wh"].T, w["ws"].T,
              w["bs"][None, :], w["wv"].T)

    def fn(hs, hv, g, b, whT, wsT, bs, wvT, out):
        s = _ln(hs[...], g[...], b[...])
        v = _vln([hv[:, 3 * d:3 * d + 3] for d in range(3)])
        s, v = _gvp(s, v, whT[...], wsT[...], bs[...], wvT[...], False, False)
        out[...] = _pack_row(s, v, ROW_T)

    return _tc_map(fn, h_s.shape[0], BN, [h_s, h_v9], consts, [ROW_T])[0]


def _embed_edges(h_s, ev3, params):
    lnp, w = params["We_ln"], params["We"]
    consts = (lnp["g"][None, :], lnp["b"][None, :], w["wh"].T, w["ws"].T,
              w["bs"][None, :], w["wv"].T)

    def fn(hs, hv, g, b, whT, wsT, bs, wvT, es_out, ev_out):
        s = _ln(hs[...], g[...], b[...])
        v = _vln([hv[:, d:d + 1] for d in range(3)])
        s, v = _gvp(s, v, whT[...], wsT[...], bs[...], wvT[...], False, False)
        es_out[...] = s
        ev_out[...] = jnp.concatenate(
            [v[0], v[1], v[2], jnp.zeros((s.shape[0], 5), jnp.float32)], axis=1)

    es, ev8 = _tc_map(fn, h_s.shape[0], BE, [h_s, ev3], consts, [32, 8])
    return es, ev8


def _tc_messages(gs, gd, es, ev8, lp):
    consts = []
    for m in ("m0", "m1", "m2"):
        consts += [lp[m]["wh"].T, lp[m]["ws"].T, lp[m]["bs"][None, :], lp[m]["wv"].T]
    consts = tuple(consts)

    def fn(gs_r, gd_r, es_r, ev_r, whT0, wsT0, bs0, wvT0, whT1, wsT1, bs1, wvT1,
           whT2, wsT2, bs2, wvT2, out):
        ss, vs = _unpack_row(gs_r[...])
        sd, vd = _unpack_row(gd_r[...])
        es_b = es_r[...]
        ev = ev_r[...]
        ms = jnp.concatenate([ss, es_b, sd], axis=1)
        mv = [jnp.concatenate([vs[d], ev[:, d:d + 1], vd[d]], axis=1) for d in range(3)]
        ms, mv = _gvp(ms, mv, whT0[...], wsT0[...], bs0[...], wvT0[...], True, True)
        ms, mv = _gvp(ms, mv, whT1[...], wsT1[...], bs1[...], wvT1[...], True, True)
        ms, mv = _gvp(ms, mv, whT2[...], wsT2[...], bs2[...], wvT2[...], False, False)
        out[...] = _pack_row(ms, mv, ROW_M)

    return _tc_map(fn, gs.shape[0], BE, [gs, gd, es, ev8], consts, [ROW_M])[0]


def _tc_node_update(t, d0, d1, lp):
    consts = (lp["n0"]["g"][None, :], lp["n0"]["b"][None, :],
              lp["f0"]["wh"].T, lp["f0"]["ws"].T, lp["f0"]["bs"][None, :], lp["f0"]["wv"].T,
              lp["f1"]["wh"].T, lp["f1"]["ws"].T, lp["f1"]["bs"][None, :], lp["f1"]["wv"].T,
              lp["n1"]["g"][None, :], lp["n1"]["b"][None, :])

    def fn(t_r, d0_r, d1_r, g0, b0, whT0, wsT0, bs0, wvT0, whT1, wsT1, bs1, wvT1,
           g1, b1, out):
        s, v = _unpack_row(t_r[...])
        dd0 = d0_r[...]
        dd1 = d1_r[...]
        cnt = jnp.maximum(dd0[:, 160:161] + dd1[:, 160:161], 1.0)
        inv = 1.0 / cnt
        s = s + (dd0[:, 0:100] + dd1[:, 0:100]) * inv
        v = [v[d] + (dd0[:, 100 + 16 * d:116 + 16 * d]
                     + dd1[:, 100 + 16 * d:116 + 16 * d]) * inv for d in range(3)]
        s = _ln(s, g0[...], b0[...])
        v = _vln(v)
        fs, fv = _gvp(s, v, whT0[...], wsT0[...], bs0[...], wvT0[...], True, True)
        fs, fv = _gvp(fs, fv, whT1[...], wsT1[...], bs1[...], wvT1[...], False, False)
        s = _ln(s + fs, g1[...], b1[...])
        v = _vln([v[d] + fv[d] for d in range(3)])
        out[...] = _pack_row(s, v, ROW_T)

    return _tc_map(fn, t.shape[0], BN, [t, d0, d1], consts, [ROW_T])[0]


def _tc_out(t, params):
    lnp, w = params["Wout_ln"], params["Wout"]
    consts = (lnp["g"][None, :], lnp["b"][None, :], w["wh"].T, w["ws"].T,
              w["bs"][None, :])

    def fn(t_r, g, b, whT, wsT, bs, out):
        s, v = _unpack_row(t_r[...])
        s = _ln(s, g[...], b[...])
        v = _vln(v)
        s, _ = _gvp(s, v, whT[...], wsT[...], bs[...], None, True, False)
        out[...] = s

    return _tc_map(fn, t.shape[0], BN, [t], consts, [100])[0]


def _sc_gather_body(table_h, src_h, dst_h, gs_h, gd_h, si_v, di_v, bs_v, bd_v,
                    s_sem, d_sem, *, tot_ch):
    wid = lax.axis_index("s") * 2 + lax.axis_index("c")
    n_ch = (tot_ch - wid + NW - 1) // NW

    def body(i, carry):
        base = (wid + i * NW) * GCH
        pltpu.sync_copy(src_h.at[pl.ds(base, GCH)], si_v)
        pltpu.sync_copy(dst_h.at[pl.ds(base, GCH)], di_v)
        cs = pltpu.async_copy(table_h.at[si_v], bs_v, s_sem)
        cd = pltpu.async_copy(table_h.at[di_v], bd_v, d_sem)
        cs.wait()
        cd.wait()
        pltpu.sync_copy(bs_v, gs_h.at[pl.ds(base, GCH)])
        pltpu.sync_copy(bd_v, gd_h.at[pl.ds(base, GCH)])
        return carry

    lax.fori_loop(0, n_ch, body, 0)


def _sc_gather(table, src, dst):
    e = src.shape[0]
    tot_ch = e // GCH
    mesh = plsc.VectorSubcoreMesh(core_axis_name="c", subcore_axis_name="s")
    kfn = pl.kernel(
        functools.partial(_sc_gather_body, tot_ch=tot_ch),
        mesh=mesh,
        out_type=[jax.ShapeDtypeStruct((e, ROW_T), jnp.float32),
                  jax.ShapeDtypeStruct((e, ROW_T), jnp.float32)],
        scratch_types=[pltpu.VMEM((GCH,), jnp.int32),
                       pltpu.VMEM((GCH,), jnp.int32),
                       pltpu.VMEM((GCH, ROW_T), jnp.float32),
                       pltpu.VMEM((GCH, ROW_T), jnp.float32),
                       pltpu.SemaphoreType.DMA,
                       pltpu.SemaphoreType.DMA],
        compiler_params=pltpu.CompilerParams(use_tc_tiling_on_sc=False),
    )
    return kfn(table, src, dst)


def _sc_scatter_body(msgs_h, dst_h, z_h, out_h, buf_v, di_v, acc_s,
                     *, tot_ch, rps):
    cid = lax.axis_index("c")
    sid = lax.axis_index("s")
    wid = sid * 2 + cid
    n_ch = (tot_ch - wid + NW - 1) // NW
    pltpu.sync_copy(z_h.at[pl.ds(sid * rps, rps)], acc_s.at[pl.ds(sid * rps, rps)])
    plsc.subcore_barrier()

    def body(i, carry):
        base = (wid + i * NW) * SCH
        pltpu.sync_copy(msgs_h.at[pl.ds(base, SCH)], buf_v)
        pltpu.sync_copy(dst_h.at[pl.ds(base, SCH)], di_v)
        pltpu.sync_copy(buf_v, acc_s.at[di_v], add=True)
        return carry

    lax.fori_loop(0, n_ch, body, 0)
    plsc.subcore_barrier()
    pltpu.sync_copy(acc_s.at[pl.ds(sid * rps, rps)],
                    out_h.at[cid, pl.ds(sid * rps, rps)])


def _sc_scatter(msgs, dst, zrows):
    e = msgs.shape[0]
    n = zrows.shape[0]
    tot_ch = e // SCH
    rps = n // 16
    mesh = plsc.VectorSubcoreMesh(core_axis_name="c", subcore_axis_name="s")
    kfn = pl.kernel(
        functools.partial(_sc_scatter_body, tot_ch=tot_ch, rps=rps),
        mesh=mesh,
        out_type=jax.ShapeDtypeStruct((2, n, ROW_M), jnp.float32),
        scratch_types=[pltpu.VMEM((SCH, ROW_M), jnp.float32),
                       pltpu.VMEM((SCH,), jnp.int32),
                       pltpu.VMEM_SHARED((n, ROW_M), jnp.float32)],
        compiler_params=pltpu.CompilerParams(use_tc_tiling_on_sc=False),
    )
    return kfn(msgs, dst, zrows)


def kernel(h_V_s, h_V_v, edge_index, h_E_s, h_E_v, batch, params):
    n = h_V_s.shape[0]
    src = edge_index[0]
    dst = edge_index[1]
    hv9 = jnp.swapaxes(h_V_v, 1, 2).reshape(n, 9)
    ev3 = h_E_v[:, 0, :]

    t = _embed_nodes(h_V_s, hv9, params)
    es, ev8 = _embed_edges(h_E_s, ev3, params)
    zrows = jnp.zeros((n, ROW_M), jnp.float32)

    for lp in params["layers"]:
        gs, gd = _sc_gather(t, src, dst)
        m = _tc_messages(gs, gd, es, ev8, lp)
        d = _sc_scatter(m, dst, zrows)
        t = _tc_node_update(t, d[0], d[1], lp)

    return _tc_out(t, params)
